# baseline (speedup 1.0000x reference)
import functools

import jax
import jax.numpy as jnp
from jax import lax
from jax.experimental import pallas as pl
from jax.experimental.pallas import tpu as pltpu

N = 10000
E = 160000
D = 256
NCHUNK = 4
FC = D // NCHUNK
R = 400
GRID = N // R

_EPS_MSG = 1e-7
_EPS_DEN = 1e-16
_EPS_LN = 1e-5


def _layer_norm(h):
    mu = jnp.mean(h, axis=-1, keepdims=True)
    var = jnp.mean((h - mu) ** 2, axis=-1, keepdims=True)
    return (h - mu) * lax.rsqrt(var + _EPS_LN)


def _write_pq(pq_ref, z, t):
    msg = jax.nn.relu(z) + _EPS_MSG
    p = jnp.exp(t * msg)
    q = p * msg
    for c in range(NCHUNK):
        pq_ref[c] = jnp.concatenate(
            [p[:, c * FC:(c + 1) * FC], q[:, c * FC:(c + 1) * FC]], axis=-1)


def _enc_body(x_ref, w_ref, b_ref, t_ref, h_ref, pq_ref):
    h = jnp.dot(x_ref[...], w_ref[...],
                preferred_element_type=jnp.float32) + b_ref[...]
    h_ref[...] = h
    _write_pq(pq_ref, h, t_ref[0, 0])


def _agg_from_nd(nd_ref):
    num = jnp.concatenate([nd_ref[c, :, FC:] for c in range(NCHUNK)], axis=-1)
    den = jnp.concatenate([nd_ref[c, :, :FC] for c in range(NCHUNK)], axis=-1)
    return num / (den + _EPS_DEN)


def _layer_body(nd_ref, z_ref, hprev_ref, w_ref, b_ref, t_ref,
                h_ref, z2_ref, pq_ref, *, first, last):
    agg = _agg_from_nd(nd_ref)
    conv = jnp.dot(agg + z_ref[...], w_ref[...],
                   preferred_element_type=jnp.float32) + b_ref[...]
    h = conv if first else hprev_ref[...] + conv
    hn = jax.nn.relu(_layer_norm(h))
    if last:
        m = jnp.max(hn, axis=-1, keepdims=True)
        ex = jnp.exp(hn - m)
        h_ref[...] = hn - m - jnp.log(jnp.sum(ex, axis=-1, keepdims=True))
    else:
        h_ref[...] = h
        z2_ref[...] = hn
        _write_pq(pq_ref, hn, t_ref[0, 0])


def _row_spec():
    return pl.BlockSpec((R, D), lambda i: (i, 0))


def _nd_spec():
    return pl.BlockSpec((NCHUNK, R, 2 * FC), lambda i: (0, i, 0))


def _full(shape):
    return pl.BlockSpec(shape, lambda i: tuple(0 for _ in shape))


_F32 = jnp.float32


def _enc_stage(x, w, b, t):
    return pl.pallas_call(
        _enc_body,
        grid=(GRID,),
        in_specs=[_row_spec(), _full((D, D)), _full((1, D)), _full((1, 1))],
        out_specs=[_row_spec(), _nd_spec()],
        out_shape=[jax.ShapeDtypeStruct((N, D), _F32),
                   jax.ShapeDtypeStruct((NCHUNK, N, 2 * FC), _F32)],
    )(x, w, b.reshape(1, D), t.reshape(1, 1))


def _layer_stage(nd, z, hprev, w, b, t_next, *, first, last):
    body = functools.partial(_layer_body, first=first, last=last)
    if first:
        def body2(nd_ref, z_ref, w_ref, b_ref, t_ref, h_ref, z2_ref, pq_ref):
            return body(nd_ref, z_ref, None, w_ref, b_ref, t_ref,
                        h_ref, z2_ref, pq_ref)
        in_specs = [_nd_spec(), _row_spec(), _full((D, D)), _full((1, D)),
                    _full((1, 1))]
        args = (nd, z, w, b.reshape(1, D), t_next.reshape(1, 1))
    elif last:
        def body2(nd_ref, z_ref, hp_ref, w_ref, b_ref, h_ref):
            return body(nd_ref, z_ref, hp_ref, w_ref, b_ref, None,
                        h_ref, None, None)
        in_specs = [_nd_spec(), _row_spec(), _row_spec(), _full((D, D)),
                    _full((1, D))]
        args = (nd, z, hprev, w, b.reshape(1, D))
    else:
        def body2(nd_ref, z_ref, hp_ref, w_ref, b_ref, t_ref,
                  h_ref, z2_ref, pq_ref):
            return body(nd_ref, z_ref, hp_ref, w_ref, b_ref, t_ref,
                        h_ref, z2_ref, pq_ref)
        in_specs = [_nd_spec(), _row_spec(), _row_spec(), _full((D, D)),
                    _full((1, D)), _full((1, 1))]
        args = (nd, z, hprev, w, b.reshape(1, D), t_next.reshape(1, 1))

    if last:
        out_specs = [_row_spec()]
        out_shape = [jax.ShapeDtypeStruct((N, D), _F32)]
    else:
        out_specs = [_row_spec(), _row_spec(), _nd_spec()]
        out_shape = [jax.ShapeDtypeStruct((N, D), _F32),
                     jax.ShapeDtypeStruct((N, D), _F32),
                     jax.ShapeDtypeStruct((NCHUNK, N, 2 * FC), _F32)]
    return pl.pallas_call(
        body2, grid=(GRID,), in_specs=in_specs, out_specs=out_specs,
        out_shape=out_shape,
    )(*args)


def _aggregate(pq, src, dst):
    rows = pq[:, src, :]
    return jax.vmap(
        lambda tbl: jax.ops.segment_sum(tbl, dst, num_segments=N))(rows)


def kernel(x, edge_index, W_enc, b_enc, W_mlp, b_mlp, t):
    src = edge_index[0]
    dst = edge_index[1]
    h0, pq = _enc_stage(x, W_enc, b_enc, t[0])
    nd = _aggregate(pq, src, dst)
    h, z, pq = _layer_stage(nd, h0, None, W_mlp[0], b_mlp[0], t[1],
                            first=True, last=False)
    for i in (1, 2):
        nd = _aggregate(pq, src, dst)
        h, z, pq = _layer_stage(nd, z, h, W_mlp[i], b_mlp[i], t[i + 1],
                                first=False, last=False)
    nd = _aggregate(pq, src, dst)
    (y,) = _layer_stage(nd, z, h, W_mlp[3], b_mlp[3], None,
                        first=False, last=True)
    return y


# speedup vs baseline: 32.2058x; 32.2058x over previous
import functools

import jax
import jax.numpy as jnp
from jax import lax
from jax.experimental import pallas as pl
from jax.experimental.pallas import tpu as pltpu
from jax.experimental.pallas import tpu_sc as plsc

N = 10000
E = 160000
D = 256
NCHUNK = 4
FC = D // NCHUNK
R = 400
GRID = N // R

_EPS_MSG = 1e-7
_EPS_DEN = 1e-16
_EPS_LN = 1e-5


def _layer_norm(h):
    mu = jnp.mean(h, axis=-1, keepdims=True)
    var = jnp.mean((h - mu) ** 2, axis=-1, keepdims=True)
    return (h - mu) * lax.rsqrt(var + _EPS_LN)


def _write_pq(pq_ref, z, t):
    msg = jax.nn.relu(z) + _EPS_MSG
    p = jnp.exp(t * msg)
    q = p * msg
    for c in range(NCHUNK):
        pq_ref[c] = jnp.concatenate(
            [p[:, c * FC:(c + 1) * FC], q[:, c * FC:(c + 1) * FC]], axis=-1)


def _enc_body(x_ref, w_ref, b_ref, t_ref, h_ref, pq_ref):
    h = jnp.dot(x_ref[...], w_ref[...],
                preferred_element_type=jnp.float32) + b_ref[...]
    h_ref[...] = h
    _write_pq(pq_ref, h, t_ref[0, 0])


def _agg_from_nd(nd_ref):
    num = jnp.concatenate([nd_ref[c, :, FC:] for c in range(NCHUNK)], axis=-1)
    den = jnp.concatenate([nd_ref[c, :, :FC] for c in range(NCHUNK)], axis=-1)
    return num / (den + _EPS_DEN)


def _layer_body(nd_ref, z_ref, hprev_ref, w_ref, b_ref, t_ref,
                h_ref, z2_ref, pq_ref, *, first, last):
    agg = _agg_from_nd(nd_ref)
    conv = jnp.dot(agg + z_ref[...], w_ref[...],
                   preferred_element_type=jnp.float32) + b_ref[...]
    h = conv if first else hprev_ref[...] + conv
    hn = jax.nn.relu(_layer_norm(h))
    if last:
        m = jnp.max(hn, axis=-1, keepdims=True)
        ex = jnp.exp(hn - m)
        h_ref[...] = hn - m - jnp.log(jnp.sum(ex, axis=-1, keepdims=True))
    else:
        h_ref[...] = h
        z2_ref[...] = hn
        _write_pq(pq_ref, hn, t_ref[0, 0])


def _row_spec():
    return pl.BlockSpec((R, D), lambda i: (i, 0))


def _nd_spec():
    return pl.BlockSpec((NCHUNK, R, 2 * FC), lambda i: (0, i, 0))


def _full(shape):
    return pl.BlockSpec(shape, lambda i: tuple(0 for _ in shape))


_F32 = jnp.float32


def _enc_stage(x, w, b, t):
    return pl.pallas_call(
        _enc_body,
        grid=(GRID,),
        in_specs=[_row_spec(), _full((D, D)), _full((1, D)), _full((1, 1))],
        out_specs=[_row_spec(), _nd_spec()],
        out_shape=[jax.ShapeDtypeStruct((N, D), _F32),
                   jax.ShapeDtypeStruct((NCHUNK, N, 2 * FC), _F32)],
    )(x, w, b.reshape(1, D), t.reshape(1, 1))


def _layer_stage(nd, z, hprev, w, b, t_next, *, first, last):
    body = functools.partial(_layer_body, first=first, last=last)
    if first:
        def body2(nd_ref, z_ref, w_ref, b_ref, t_ref, h_ref, z2_ref, pq_ref):
            return body(nd_ref, z_ref, None, w_ref, b_ref, t_ref,
                        h_ref, z2_ref, pq_ref)
        in_specs = [_nd_spec(), _row_spec(), _full((D, D)), _full((1, D)),
                    _full((1, 1))]
        args = (nd, z, w, b.reshape(1, D), t_next.reshape(1, 1))
    elif last:
        def body2(nd_ref, z_ref, hp_ref, w_ref, b_ref, h_ref):
            return body(nd_ref, z_ref, hp_ref, w_ref, b_ref, None,
                        h_ref, None, None)
        in_specs = [_nd_spec(), _row_spec(), _row_spec(), _full((D, D)),
                    _full((1, D))]
        args = (nd, z, hprev, w, b.reshape(1, D))
    else:
        def body2(nd_ref, z_ref, hp_ref, w_ref, b_ref, t_ref,
                  h_ref, z2_ref, pq_ref):
            return body(nd_ref, z_ref, hp_ref, w_ref, b_ref, t_ref,
                        h_ref, z2_ref, pq_ref)
        in_specs = [_nd_spec(), _row_spec(), _row_spec(), _full((D, D)),
                    _full((1, D)), _full((1, 1))]
        args = (nd, z, hprev, w, b.reshape(1, D), t_next.reshape(1, 1))

    if last:
        out_specs = [_row_spec()]
        out_shape = [jax.ShapeDtypeStruct((N, D), _F32)]
    else:
        out_specs = [_row_spec(), _row_spec(), _nd_spec()]
        out_shape = [jax.ShapeDtypeStruct((N, D), _F32),
                     jax.ShapeDtypeStruct((N, D), _F32),
                     jax.ShapeDtypeStruct((NCHUNK, N, 2 * FC), _F32)]
    return pl.pallas_call(
        body2, grid=(GRID,), in_specs=in_specs, out_specs=out_specs,
        out_shape=out_shape,
    )(*args)


_SC_CORES = 2
_SC_TILES = 16
_B = 128
_EPT = E // _SC_TILES
_NB = _EPT // _B
_TAIL = _EPT - _NB * _B
_RPT = 624
_RPT_LAST = N - (_SC_TILES - 1) * _RPT
_CPC = NCHUNK // _SC_CORES

_sc_mesh = plsc.VectorSubcoreMesh(core_axis_name="c", subcore_axis_name="s")


@functools.partial(
    pl.kernel,
    out_type=jax.ShapeDtypeStruct((NCHUNK * N, 2 * FC), jnp.float32),
    mesh=_sc_mesh,
    scratch_types=[
        pltpu.VMEM_SHARED((N, 2 * FC), jnp.float32),
        pltpu.VMEM((_B, 2 * FC), jnp.float32),
        pltpu.VMEM((_B,), jnp.int32),
        pltpu.VMEM((_B,), jnp.int32),
        pltpu.VMEM((_TAIL, 2 * FC), jnp.float32),
        pltpu.VMEM((_TAIL,), jnp.int32),
        pltpu.VMEM((_TAIL,), jnp.int32),
        pltpu.VMEM((16, 2 * FC), jnp.float32),
    ],
)
def _sc_agg(pq_hbm, src4_hbm, dst_hbm, out_hbm,
            acc, rows, srcv, dstv, rows_t, srcv_t, dstv_t, zbuf):
    cid = lax.axis_index("c")
    sid = lax.axis_index("s")
    zero16 = jnp.zeros((16,), jnp.float32)

    def _zero_row(i, carry):
        for j in range(2 * FC // 16):
            zbuf[i, pl.ds(j * 16, 16)] = zero16
        return carry
    lax.fori_loop(0, 16, _zero_row, 0)

    n_strips = 39 + jnp.where(sid == _SC_TILES - 1, 1, 0)

    for jchunk in range(_CPC):
        c = cid * _CPC + jchunk

        def _zero_strip(i, carry):
            pltpu.sync_copy(zbuf, acc.at[pl.ds(sid * _RPT + i * 16, 16)])
            return carry
        lax.fori_loop(0, n_strips, _zero_strip, 0)

        plsc.subcore_barrier()

        def _block(k, carry):
            base = sid * _EPT + k * _B
            pltpu.sync_copy(src4_hbm.at[pl.ds(c * E + base, _B)], srcv)
            pltpu.sync_copy(dst_hbm.at[pl.ds(base, _B)], dstv)
            pltpu.sync_copy(pq_hbm.at[srcv], rows)
            pltpu.sync_copy(rows, acc.at[dstv], add=True)
            return carry
        lax.fori_loop(0, _NB, _block, 0)

        base = sid * _EPT + _NB * _B
        pltpu.sync_copy(src4_hbm.at[pl.ds(c * E + base, _TAIL)], srcv_t)
        pltpu.sync_copy(dst_hbm.at[pl.ds(base, _TAIL)], dstv_t)
        pltpu.sync_copy(pq_hbm.at[srcv_t], rows_t)
        pltpu.sync_copy(rows_t, acc.at[dstv_t], add=True)

        plsc.subcore_barrier()

        @pl.when(sid < _SC_TILES - 1)
        def _():
            pltpu.sync_copy(acc.at[pl.ds(sid * _RPT, _RPT)],
                            out_hbm.at[pl.ds(c * N + sid * _RPT, _RPT)])

        @pl.when(sid == _SC_TILES - 1)
        def _():
            pltpu.sync_copy(acc.at[pl.ds(sid * _RPT, _RPT_LAST)],
                            out_hbm.at[pl.ds(c * N + sid * _RPT, _RPT_LAST)])

        if jchunk + 1 < _CPC:
            plsc.subcore_barrier()


def _aggregate(pq, src4, dst):
    nd_flat = _sc_agg(pq.reshape(NCHUNK * N, 2 * FC), src4, dst)
    return nd_flat.reshape(NCHUNK, N, 2 * FC)


def kernel(x, edge_index, W_enc, b_enc, W_mlp, b_mlp, t):
    src = edge_index[0]
    dst = edge_index[1]
    src4 = (jnp.arange(NCHUNK, dtype=jnp.int32)[:, None] * N
            + src[None, :]).reshape(-1)
    h0, pq = _enc_stage(x, W_enc, b_enc, t[0])
    nd = _aggregate(pq, src4, dst)
    h, z, pq = _layer_stage(nd, h0, None, W_mlp[0], b_mlp[0], t[1],
                            first=True, last=False)
    for i in (1, 2):
        nd = _aggregate(pq, src4, dst)
        h, z, pq = _layer_stage(nd, z, h, W_mlp[i], b_mlp[i], t[i + 1],
                                first=False, last=False)
    nd = _aggregate(pq, src4, dst)
    (y,) = _layer_stage(nd, z, h, W_mlp[3], b_mlp[3], None,
                        first=False, last=True)
    return y


# speedup vs baseline: 56.0188x; 1.7394x over previous
import functools

import jax
import jax.numpy as jnp
from jax import lax
from jax.experimental import pallas as pl
from jax.experimental.pallas import tpu as pltpu
from jax.experimental.pallas import tpu_sc as plsc

N = 10000
E = 160000
D = 256
NCHUNK = 4
FC = D // NCHUNK
R = 400
GRID = N // R

_EPS_MSG = 1e-7
_EPS_DEN = 1e-16
_EPS_LN = 1e-5


def _layer_norm(h):
    mu = jnp.mean(h, axis=-1, keepdims=True)
    var = jnp.mean((h - mu) ** 2, axis=-1, keepdims=True)
    return (h - mu) * lax.rsqrt(var + _EPS_LN)


def _write_pq(pq_ref, z, t):
    msg = jax.nn.relu(z) + _EPS_MSG
    p = jnp.exp(t * msg)
    q = p * msg
    for c in range(NCHUNK):
        pq_ref[c] = jnp.concatenate(
            [p[:, c * FC:(c + 1) * FC], q[:, c * FC:(c + 1) * FC]], axis=-1)


def _enc_body(x_ref, w_ref, b_ref, t_ref, h_ref, pq_ref):
    h = jnp.dot(x_ref[...], w_ref[...],
                preferred_element_type=jnp.float32) + b_ref[...]
    h_ref[...] = h
    _write_pq(pq_ref, h, t_ref[0, 0])


def _agg_from_nd(nd_ref):
    num = jnp.concatenate([nd_ref[c, :, FC:] for c in range(NCHUNK)], axis=-1)
    den = jnp.concatenate([nd_ref[c, :, :FC] for c in range(NCHUNK)], axis=-1)
    return num / (den + _EPS_DEN)


def _layer_body(nd_ref, z_ref, hprev_ref, w_ref, b_ref, t_ref,
                h_ref, z2_ref, pq_ref, *, first, last):
    agg = _agg_from_nd(nd_ref)
    conv = jnp.dot(agg + z_ref[...], w_ref[...],
                   preferred_element_type=jnp.float32) + b_ref[...]
    h = conv if first else hprev_ref[...] + conv
    hn = jax.nn.relu(_layer_norm(h))
    if last:
        m = jnp.max(hn, axis=-1, keepdims=True)
        ex = jnp.exp(hn - m)
        h_ref[...] = hn - m - jnp.log(jnp.sum(ex, axis=-1, keepdims=True))
    else:
        h_ref[...] = h
        z2_ref[...] = hn
        _write_pq(pq_ref, hn, t_ref[0, 0])


def _row_spec():
    return pl.BlockSpec((R, D), lambda i: (i, 0))


def _nd_spec():
    return pl.BlockSpec((NCHUNK, R, 2 * FC), lambda i: (0, i, 0))


def _full(shape):
    return pl.BlockSpec(shape, lambda i: tuple(0 for _ in shape))


_F32 = jnp.float32


def _enc_stage(x, w, b, t):
    return pl.pallas_call(
        _enc_body,
        grid=(GRID,),
        in_specs=[_row_spec(), _full((D, D)), _full((1, D)), _full((1, 1))],
        out_specs=[_row_spec(), _nd_spec()],
        out_shape=[jax.ShapeDtypeStruct((N, D), _F32),
                   jax.ShapeDtypeStruct((NCHUNK, N, 2 * FC), _F32)],
    )(x, w, b.reshape(1, D), t.reshape(1, 1))


def _layer_stage(nd, z, hprev, w, b, t_next, *, first, last):
    body = functools.partial(_layer_body, first=first, last=last)
    if first:
        def body2(nd_ref, z_ref, w_ref, b_ref, t_ref, h_ref, z2_ref, pq_ref):
            return body(nd_ref, z_ref, None, w_ref, b_ref, t_ref,
                        h_ref, z2_ref, pq_ref)
        in_specs = [_nd_spec(), _row_spec(), _full((D, D)), _full((1, D)),
                    _full((1, 1))]
        args = (nd, z, w, b.reshape(1, D), t_next.reshape(1, 1))
    elif last:
        def body2(nd_ref, z_ref, hp_ref, w_ref, b_ref, h_ref):
            return body(nd_ref, z_ref, hp_ref, w_ref, b_ref, None,
                        h_ref, None, None)
        in_specs = [_nd_spec(), _row_spec(), _row_spec(), _full((D, D)),
                    _full((1, D))]
        args = (nd, z, hprev, w, b.reshape(1, D))
    else:
        def body2(nd_ref, z_ref, hp_ref, w_ref, b_ref, t_ref,
                  h_ref, z2_ref, pq_ref):
            return body(nd_ref, z_ref, hp_ref, w_ref, b_ref, t_ref,
                        h_ref, z2_ref, pq_ref)
        in_specs = [_nd_spec(), _row_spec(), _row_spec(), _full((D, D)),
                    _full((1, D)), _full((1, 1))]
        args = (nd, z, hprev, w, b.reshape(1, D), t_next.reshape(1, 1))

    if last:
        out_specs = [_row_spec()]
        out_shape = [jax.ShapeDtypeStruct((N, D), _F32)]
    else:
        out_specs = [_row_spec(), _row_spec(), _nd_spec()]
        out_shape = [jax.ShapeDtypeStruct((N, D), _F32),
                     jax.ShapeDtypeStruct((N, D), _F32),
                     jax.ShapeDtypeStruct((NCHUNK, N, 2 * FC), _F32)]
    return pl.pallas_call(
        body2, grid=(GRID,), in_specs=in_specs, out_specs=out_specs,
        out_shape=out_shape,
    )(*args)


_SC_CORES = 2
_SC_TILES = 16
_B = 128
_NBLK = E // _B
_NB = _NBLK // _SC_TILES
_RPT = 624
_RPT_LAST = N - (_SC_TILES - 1) * _RPT
_CPC = NCHUNK // _SC_CORES

_sc_mesh = plsc.VectorSubcoreMesh(core_axis_name="c", subcore_axis_name="s")


@functools.partial(
    pl.kernel,
    out_type=jax.ShapeDtypeStruct((NCHUNK * N, 2 * FC), jnp.float32),
    mesh=_sc_mesh,
    scratch_types=[
        pltpu.VMEM_SHARED((N, 2 * FC), jnp.float32),
        pltpu.VMEM((_B, 2 * FC), jnp.float32),
        pltpu.VMEM((_B, 2 * FC), jnp.float32),
        pltpu.VMEM((_B,), jnp.int32),
        pltpu.VMEM((_B,), jnp.int32),
        pltpu.VMEM((_B,), jnp.int32),
        pltpu.VMEM((_B,), jnp.int32),
        pltpu.VMEM((16, 2 * FC), jnp.float32),
        pltpu.SemaphoreType.DMA,
        pltpu.SemaphoreType.DMA,
        pltpu.SemaphoreType.DMA,
        pltpu.SemaphoreType.DMA,
    ],
)
def _sc_agg(pq_hbm, src4_hbm, dst_hbm, out_hbm,
            acc, rows0, rows1, srcv0, srcv1, dstv0, dstv1, zbuf,
            isem0, isem1, gsem0, gsem1):
    cid = lax.axis_index("c")
    sid = lax.axis_index("s")
    zero16 = jnp.zeros((16,), jnp.float32)

    def _zero_row(i, carry):
        for j in range(2 * FC // 16):
            zbuf[i, pl.ds(j * 16, 16)] = zero16
        return carry
    lax.fori_loop(0, 16, _zero_row, 0)

    n_strips = 39 + jnp.where(sid == _SC_TILES - 1, 1, 0)

    for jchunk in range(_CPC):
        c = cid * _CPC + jchunk

        def _idx_load(j, sv, dv, sem):
            g = sid + _SC_TILES * j
            pltpu.async_copy(src4_hbm.at[pl.ds(c * E + g * _B, _B)], sv, sem)
            pltpu.async_copy(dst_hbm.at[pl.ds(g * _B, _B)], dv, sem)

        def _idx_wait(sv, dv, sem):
            pltpu.make_async_copy(src4_hbm.at[pl.ds(0, _B)], sv, sem).wait()
            pltpu.make_async_copy(dst_hbm.at[pl.ds(0, _B)], dv, sem).wait()

        def _gather(sv, rows, sem):
            pltpu.async_copy(pq_hbm.at[sv], rows, sem)

        def _gather_wait(rows, sem):
            pltpu.make_async_copy(pq_hbm.at[pl.ds(0, _B)], rows, sem).wait()

        def _scatter(rows, dv):
            pltpu.sync_copy(rows, acc.at[dv], add=True)

        def _zero_strip(i, carry):
            pltpu.sync_copy(zbuf, acc.at[pl.ds(sid * _RPT + i * 16, 16)])
            return carry
        lax.fori_loop(0, n_strips, _zero_strip, 0)

        plsc.subcore_barrier()

        _idx_load(0, srcv0, dstv0, isem0)
        _idx_wait(srcv0, dstv0, isem0)
        _gather(srcv0, rows0, gsem0)
        _idx_load(1, srcv1, dstv1, isem1)

        def _pair(m, carry):
            j0 = 2 * m
            _gather_wait(rows0, gsem0)
            _idx_wait(srcv1, dstv1, isem1)
            _gather(srcv1, rows1, gsem1)
            _scatter(rows0, dstv0)

            @pl.when(m < _NB // 2 - 1)
            def _():
                _idx_load(j0 + 2, srcv0, dstv0, isem0)
            _gather_wait(rows1, gsem1)

            @pl.when(m < _NB // 2 - 1)
            def _():
                _idx_wait(srcv0, dstv0, isem0)
                _gather(srcv0, rows0, gsem0)
            _scatter(rows1, dstv1)

            @pl.when(m < _NB // 2 - 1)
            def _():
                _idx_load(j0 + 3, srcv1, dstv1, isem1)
            return carry
        lax.fori_loop(0, _NB // 2, _pair, 0)

        @pl.when(sid < _NBLK - _NB * _SC_TILES)
        def _():
            g = sid + _SC_TILES * _NB
            pltpu.sync_copy(src4_hbm.at[pl.ds(c * E + g * _B, _B)], srcv0)
            pltpu.sync_copy(dst_hbm.at[pl.ds(g * _B, _B)], dstv0)
            pltpu.sync_copy(pq_hbm.at[srcv0], rows0)
            _scatter(rows0, dstv0)

        plsc.subcore_barrier()

        @pl.when(sid < _SC_TILES - 1)
        def _():
            pltpu.sync_copy(acc.at[pl.ds(sid * _RPT, _RPT)],
                            out_hbm.at[pl.ds(c * N + sid * _RPT, _RPT)])

        @pl.when(sid == _SC_TILES - 1)
        def _():
            pltpu.sync_copy(acc.at[pl.ds(sid * _RPT, _RPT_LAST)],
                            out_hbm.at[pl.ds(c * N + sid * _RPT, _RPT_LAST)])

        if jchunk + 1 < _CPC:
            plsc.subcore_barrier()


def _aggregate(pq, src4, dst):
    nd_flat = _sc_agg(pq.reshape(NCHUNK * N, 2 * FC), src4, dst)
    return nd_flat.reshape(NCHUNK, N, 2 * FC)


def kernel(x, edge_index, W_enc, b_enc, W_mlp, b_mlp, t):
    src = edge_index[0]
    dst = edge_index[1]
    src4 = (jnp.arange(NCHUNK, dtype=jnp.int32)[:, None] * N
            + src[None, :]).reshape(-1)
    h0, pq = _enc_stage(x, W_enc, b_enc, t[0])
    nd = _aggregate(pq, src4, dst)
    h, z, pq = _layer_stage(nd, h0, None, W_mlp[0], b_mlp[0], t[1],
                            first=True, last=False)
    for i in (1, 2):
        nd = _aggregate(pq, src4, dst)
        h, z, pq = _layer_stage(nd, z, h, W_mlp[i], b_mlp[i], t[i + 1],
                                first=False, last=False)
    nd = _aggregate(pq, src4, dst)
    (y,) = _layer_stage(nd, z, h, W_mlp[3], b_mlp[3], None,
                        first=False, last=True)
    return y


# speedup vs baseline: 59.5124x; 1.0624x over previous
import functools

import jax
import jax.numpy as jnp
from jax import lax
from jax.experimental import pallas as pl
from jax.experimental.pallas import tpu as pltpu
from jax.experimental.pallas import tpu_sc as plsc

N = 10000
E = 160000
D = 256
NCHUNK = 4
FC = D // NCHUNK
R = 400
GRID = N // R

_EPS_MSG = 1e-7
_EPS_DEN = 1e-16
_EPS_LN = 1e-5


def _layer_norm(h):
    mu = jnp.mean(h, axis=-1, keepdims=True)
    var = jnp.mean((h - mu) ** 2, axis=-1, keepdims=True)
    return (h - mu) * lax.rsqrt(var + _EPS_LN)


def _write_pq(pq_ref, z, t):
    msg = jax.nn.relu(z) + _EPS_MSG
    p = jnp.exp(t * msg)
    q = p * msg
    for c in range(NCHUNK):
        pq_ref[c] = jnp.concatenate(
            [p[:, c * FC:(c + 1) * FC], q[:, c * FC:(c + 1) * FC]], axis=-1)


def _enc_body(x_ref, w_ref, b_ref, t_ref, h_ref, pq_ref):
    h = jnp.dot(x_ref[...], w_ref[...],
                preferred_element_type=jnp.float32) + b_ref[...]
    h_ref[...] = h
    _write_pq(pq_ref, h, t_ref[0, 0])


def _agg_from_nd(nd_ref):
    num = jnp.concatenate([nd_ref[c, :, FC:] for c in range(NCHUNK)], axis=-1)
    den = jnp.concatenate([nd_ref[c, :, :FC] for c in range(NCHUNK)], axis=-1)
    return num / (den + _EPS_DEN)


def _layer_body(nd_ref, z_ref, hprev_ref, w_ref, b_ref, t_ref,
                h_ref, z2_ref, pq_ref, *, first, last):
    agg = _agg_from_nd(nd_ref)
    conv = jnp.dot(agg + z_ref[...], w_ref[...],
                   preferred_element_type=jnp.float32) + b_ref[...]
    h = conv if first else hprev_ref[...] + conv
    hn = jax.nn.relu(_layer_norm(h))
    if last:
        m = jnp.max(hn, axis=-1, keepdims=True)
        ex = jnp.exp(hn - m)
        h_ref[...] = hn - m - jnp.log(jnp.sum(ex, axis=-1, keepdims=True))
    else:
        h_ref[...] = h
        z2_ref[...] = hn
        _write_pq(pq_ref, hn, t_ref[0, 0])


def _row_spec():
    return pl.BlockSpec((R, D), lambda i: (i, 0))


def _nd_spec():
    return pl.BlockSpec((NCHUNK, R, 2 * FC), lambda i: (0, i, 0))


def _full(shape):
    return pl.BlockSpec(shape, lambda i: tuple(0 for _ in shape))


_F32 = jnp.float32


def _enc_stage(x, w, b, t):
    return pl.pallas_call(
        _enc_body,
        grid=(GRID,),
        in_specs=[_row_spec(), _full((D, D)), _full((1, D)), _full((1, 1))],
        out_specs=[_row_spec(), _nd_spec()],
        out_shape=[jax.ShapeDtypeStruct((N, D), _F32),
                   jax.ShapeDtypeStruct((NCHUNK, N, 2 * FC), _F32)],
    )(x, w, b.reshape(1, D), t.reshape(1, 1))


def _layer_stage(nd, z, hprev, w, b, t_next, *, first, last):
    body = functools.partial(_layer_body, first=first, last=last)
    if first:
        def body2(nd_ref, z_ref, w_ref, b_ref, t_ref, h_ref, z2_ref, pq_ref):
            return body(nd_ref, z_ref, None, w_ref, b_ref, t_ref,
                        h_ref, z2_ref, pq_ref)
        in_specs = [_nd_spec(), _row_spec(), _full((D, D)), _full((1, D)),
                    _full((1, 1))]
        args = (nd, z, w, b.reshape(1, D), t_next.reshape(1, 1))
    elif last:
        def body2(nd_ref, z_ref, hp_ref, w_ref, b_ref, h_ref):
            return body(nd_ref, z_ref, hp_ref, w_ref, b_ref, None,
                        h_ref, None, None)
        in_specs = [_nd_spec(), _row_spec(), _row_spec(), _full((D, D)),
                    _full((1, D))]
        args = (nd, z, hprev, w, b.reshape(1, D))
    else:
        def body2(nd_ref, z_ref, hp_ref, w_ref, b_ref, t_ref,
                  h_ref, z2_ref, pq_ref):
            return body(nd_ref, z_ref, hp_ref, w_ref, b_ref, t_ref,
                        h_ref, z2_ref, pq_ref)
        in_specs = [_nd_spec(), _row_spec(), _row_spec(), _full((D, D)),
                    _full((1, D)), _full((1, 1))]
        args = (nd, z, hprev, w, b.reshape(1, D), t_next.reshape(1, 1))

    if last:
        out_specs = [_row_spec()]
        out_shape = [jax.ShapeDtypeStruct((N, D), _F32)]
    else:
        out_specs = [_row_spec(), _row_spec(), _nd_spec()]
        out_shape = [jax.ShapeDtypeStruct((N, D), _F32),
                     jax.ShapeDtypeStruct((N, D), _F32),
                     jax.ShapeDtypeStruct((NCHUNK, N, 2 * FC), _F32)]
    return pl.pallas_call(
        body2, grid=(GRID,), in_specs=in_specs, out_specs=out_specs,
        out_shape=out_shape,
    )(*args)


_SC_CORES = 2
_SC_TILES = 16
_B = 128
_NBLK = E // _B
_NB = _NBLK // _SC_TILES
_RPT = 624
_RPT_LAST = N - (_SC_TILES - 1) * _RPT
_CPC = NCHUNK // _SC_CORES

_sc_mesh = plsc.VectorSubcoreMesh(core_axis_name="c", subcore_axis_name="s")


@functools.partial(
    pl.kernel,
    out_type=jax.ShapeDtypeStruct((NCHUNK * N, 2 * FC), jnp.float32),
    mesh=_sc_mesh,
    scratch_types=[
        pltpu.VMEM_SHARED((N, 2 * FC), jnp.float32),
        pltpu.VMEM((_B, 2 * FC), jnp.float32),
        pltpu.VMEM((_B, 2 * FC), jnp.float32),
        pltpu.VMEM((_B, 2 * FC), jnp.float32),
        pltpu.VMEM((_B,), jnp.int32),
        pltpu.VMEM((_B,), jnp.int32),
        pltpu.VMEM((_B,), jnp.int32),
        pltpu.VMEM((_B,), jnp.int32),
        pltpu.VMEM((_B,), jnp.int32),
        pltpu.VMEM((_B,), jnp.int32),
        pltpu.VMEM((8, 2 * FC), jnp.float32),
        pltpu.SemaphoreType.DMA,
        pltpu.SemaphoreType.DMA,
        pltpu.SemaphoreType.DMA,
        pltpu.SemaphoreType.DMA,
        pltpu.SemaphoreType.DMA,
        pltpu.SemaphoreType.DMA,
    ],
)
def _sc_agg(pq_hbm, src4_hbm, dst_hbm, out_hbm,
            acc, rows0, rows1, rows2, srcv0, srcv1, srcv2,
            dstv0, dstv1, dstv2, zbuf, isem0, isem1, isem2,
            gsem0, gsem1, gsem2):
    cid = lax.axis_index("c")
    sid = lax.axis_index("s")
    rows = (rows0, rows1, rows2)
    srcv = (srcv0, srcv1, srcv2)
    dstv = (dstv0, dstv1, dstv2)
    isem = (isem0, isem1, isem2)
    gsem = (gsem0, gsem1, gsem2)
    zero16 = jnp.zeros((16,), jnp.float32)

    def _zero_row(i, carry):
        for j in range(2 * FC // 16):
            zbuf[i, pl.ds(j * 16, 16)] = zero16
        return carry
    lax.fori_loop(0, 8, _zero_row, 0)

    n_strips = 78 + jnp.where(sid == _SC_TILES - 1, 2, 0)

    for jchunk in range(_CPC):
        c = cid * _CPC + jchunk

        def _idx_load(j, sv, dv, sem):
            g = sid + _SC_TILES * j
            pltpu.async_copy(src4_hbm.at[pl.ds(c * E + g * _B, _B)], sv, sem)
            pltpu.async_copy(dst_hbm.at[pl.ds(g * _B, _B)], dv, sem)

        def _idx_wait(sv, dv, sem):
            pltpu.make_async_copy(src4_hbm.at[pl.ds(0, _B)], sv, sem).wait()
            pltpu.make_async_copy(dst_hbm.at[pl.ds(0, _B)], dv, sem).wait()

        def _gather(sv, rbuf, sem):
            pltpu.async_copy(pq_hbm.at[sv], rbuf, sem)

        def _gather_wait(rbuf, sem):
            pltpu.make_async_copy(pq_hbm.at[pl.ds(0, _B)], rbuf, sem).wait()

        def _scatter(rbuf, dv):
            pltpu.sync_copy(rbuf, acc.at[dv], add=True)

        def _zero_strip(i, carry):
            pltpu.sync_copy(zbuf, acc.at[pl.ds(sid * _RPT + i * 8, 8)])
            return carry
        lax.fori_loop(0, n_strips, _zero_strip, 0)

        plsc.subcore_barrier()

        _idx_load(0, srcv0, dstv0, isem0)
        _idx_load(1, srcv1, dstv1, isem1)
        _idx_load(2, srcv2, dstv2, isem2)
        _idx_wait(srcv0, dstv0, isem0)
        _gather(srcv0, rows0, gsem0)
        _idx_wait(srcv1, dstv1, isem1)
        _gather(srcv1, rows1, gsem1)

        def _step(j, b, m):
            _gather_wait(rows[b], gsem[b])
            b2 = (b + 2) % 3
            if b == 0:
                _idx_wait(srcv[b2], dstv[b2], isem[b2])
                _gather(srcv[b2], rows[b2], gsem[b2])
            else:
                @pl.when(m < _NB // 3 - 1)
                def _():
                    _idx_wait(srcv[b2], dstv[b2], isem[b2])
                    _gather(srcv[b2], rows[b2], gsem[b2])
            _scatter(rows[b], dstv[b])

            @pl.when(m < _NB // 3 - 1)
            def _():
                _idx_load(j + 3, srcv[b], dstv[b], isem[b])

        def _tri(m, carry):
            for b in range(3):
                _step(3 * m + b, b, m)
            return carry
        lax.fori_loop(0, _NB // 3, _tri, 0)

        @pl.when(sid < _NBLK - _NB * _SC_TILES)
        def _():
            g = sid + _SC_TILES * _NB
            pltpu.sync_copy(src4_hbm.at[pl.ds(c * E + g * _B, _B)], srcv0)
            pltpu.sync_copy(dst_hbm.at[pl.ds(g * _B, _B)], dstv0)
            pltpu.sync_copy(pq_hbm.at[srcv0], rows0)
            _scatter(rows0, dstv0)

        plsc.subcore_barrier()

        @pl.when(sid < _SC_TILES - 1)
        def _():
            pltpu.sync_copy(acc.at[pl.ds(sid * _RPT, _RPT)],
                            out_hbm.at[pl.ds(c * N + sid * _RPT, _RPT)])

        @pl.when(sid == _SC_TILES - 1)
        def _():
            pltpu.sync_copy(acc.at[pl.ds(sid * _RPT, _RPT_LAST)],
                            out_hbm.at[pl.ds(c * N + sid * _RPT, _RPT_LAST)])

        if jchunk + 1 < _CPC:
            plsc.subcore_barrier()


def _aggregate(pq, src4, dst):
    nd_flat = _sc_agg(pq.reshape(NCHUNK * N, 2 * FC), src4, dst)
    return nd_flat.reshape(NCHUNK, N, 2 * FC)


def kernel(x, edge_index, W_enc, b_enc, W_mlp, b_mlp, t):
    src = edge_index[0]
    dst = edge_index[1]
    src4 = (jnp.arange(NCHUNK, dtype=jnp.int32)[:, None] * N
            + src[None, :]).reshape(-1)
    h0, pq = _enc_stage(x, W_enc, b_enc, t[0])
    nd = _aggregate(pq, src4, dst)
    h, z, pq = _layer_stage(nd, h0, None, W_mlp[0], b_mlp[0], t[1],
                            first=True, last=False)
    for i in (1, 2):
        nd = _aggregate(pq, src4, dst)
        h, z, pq = _layer_stage(nd, z, h, W_mlp[i], b_mlp[i], t[i + 1],
                                first=False, last=False)
    nd = _aggregate(pq, src4, dst)
    (y,) = _layer_stage(nd, z, h, W_mlp[3], b_mlp[3], None,
                        first=False, last=True)
    return y


# speedup vs baseline: 59.8637x; 1.0059x over previous
import functools

import jax
import jax.numpy as jnp
from jax import lax
from jax.experimental import pallas as pl
from jax.experimental.pallas import tpu as pltpu
from jax.experimental.pallas import tpu_sc as plsc

N = 10000
E = 160000
D = 256
NCHUNK = 4
FC = D // NCHUNK
R = 400
GRID = N // R

_EPS_MSG = 1e-7
_EPS_DEN = 1e-16
_EPS_LN = 1e-5


def _layer_norm(h):
    mu = jnp.mean(h, axis=-1, keepdims=True)
    var = jnp.mean((h - mu) ** 2, axis=-1, keepdims=True)
    return (h - mu) * lax.rsqrt(var + _EPS_LN)


def _write_pq(pq_ref, z, t):
    msg = jax.nn.relu(z) + _EPS_MSG
    p = jnp.exp(t * msg)
    q = p * msg
    for c in range(NCHUNK):
        pq_ref[c] = jnp.concatenate(
            [p[:, c * FC:(c + 1) * FC], q[:, c * FC:(c + 1) * FC]], axis=-1)


def _enc_body(x_ref, w_ref, b_ref, t_ref, h_ref, pq_ref):
    h = jnp.dot(x_ref[...], w_ref[...],
                preferred_element_type=jnp.float32) + b_ref[...]
    h_ref[...] = h
    _write_pq(pq_ref, h, t_ref[0, 0])


def _agg_from_nd(nd_ref):
    num = jnp.concatenate([nd_ref[c, :, FC:] for c in range(NCHUNK)], axis=-1)
    den = jnp.concatenate([nd_ref[c, :, :FC] for c in range(NCHUNK)], axis=-1)
    return num / (den + _EPS_DEN)


def _layer_body(nd_ref, hprev_ref, w_ref, b_ref, t_ref,
                h_ref, pq_ref, *, first, last):
    hp = hprev_ref[...]
    z = hp if first else jax.nn.relu(_layer_norm(hp))
    agg = _agg_from_nd(nd_ref)
    conv = jnp.dot(agg + z, w_ref[...],
                   preferred_element_type=jnp.float32) + b_ref[...]
    h = conv if first else hp + conv
    hn = jax.nn.relu(_layer_norm(h))
    if last:
        m = jnp.max(hn, axis=-1, keepdims=True)
        ex = jnp.exp(hn - m)
        h_ref[...] = hn - m - jnp.log(jnp.sum(ex, axis=-1, keepdims=True))
    else:
        h_ref[...] = h
        _write_pq(pq_ref, hn, t_ref[0, 0])


def _row_spec():
    return pl.BlockSpec((R, D), lambda i: (i, 0))


def _nd_spec():
    return pl.BlockSpec((NCHUNK, R, 2 * FC), lambda i: (0, i, 0))


def _full(shape):
    return pl.BlockSpec(shape, lambda i: tuple(0 for _ in shape))


_F32 = jnp.float32


def _enc_stage(x, w, b, t):
    return pl.pallas_call(
        _enc_body,
        grid=(GRID,),
        in_specs=[_row_spec(), _full((D, D)), _full((1, D)), _full((1, 1))],
        out_specs=[_row_spec(), _nd_spec()],
        out_shape=[jax.ShapeDtypeStruct((N, D), _F32),
                   jax.ShapeDtypeStruct((NCHUNK, N, 2 * FC), _F32)],
    )(x, w, b.reshape(1, D), t.reshape(1, 1))


def _layer_stage(nd, hprev, w, b, t_next, *, first, last):
    body = functools.partial(_layer_body, first=first, last=last)
    if last:
        def body2(nd_ref, hp_ref, w_ref, b_ref, h_ref):
            return body(nd_ref, hp_ref, w_ref, b_ref, None, h_ref, None)
        in_specs = [_nd_spec(), _row_spec(), _full((D, D)), _full((1, D))]
        args = (nd, hprev, w, b.reshape(1, D))
        out_specs = [_row_spec()]
        out_shape = [jax.ShapeDtypeStruct((N, D), _F32)]
    else:
        def body2(nd_ref, hp_ref, w_ref, b_ref, t_ref, h_ref, pq_ref):
            return body(nd_ref, hp_ref, w_ref, b_ref, t_ref, h_ref, pq_ref)
        in_specs = [_nd_spec(), _row_spec(), _full((D, D)), _full((1, D)),
                    _full((1, 1))]
        args = (nd, hprev, w, b.reshape(1, D), t_next.reshape(1, 1))
        out_specs = [_row_spec(), _nd_spec()]
        out_shape = [jax.ShapeDtypeStruct((N, D), _F32),
                     jax.ShapeDtypeStruct((NCHUNK, N, 2 * FC), _F32)]
    return pl.pallas_call(
        body2, grid=(GRID,), in_specs=in_specs, out_specs=out_specs,
        out_shape=out_shape,
    )(*args)


_SC_CORES = 2
_SC_TILES = 16
_B = 128
_NBLK = E // _B
_NB = _NBLK // _SC_TILES
_RPT = 624
_RPT_LAST = N - (_SC_TILES - 1) * _RPT
_CPC = NCHUNK // _SC_CORES

_sc_mesh = plsc.VectorSubcoreMesh(core_axis_name="c", subcore_axis_name="s")


@functools.partial(
    pl.kernel,
    out_type=jax.ShapeDtypeStruct((NCHUNK * N, 2 * FC), jnp.float32),
    mesh=_sc_mesh,
    scratch_types=[
        pltpu.VMEM_SHARED((N, 2 * FC), jnp.float32),
        pltpu.VMEM((_B, 2 * FC), jnp.float32),
        pltpu.VMEM((_B, 2 * FC), jnp.float32),
        pltpu.VMEM((_B, 2 * FC), jnp.float32),
        pltpu.VMEM((_B,), jnp.int32),
        pltpu.VMEM((_B,), jnp.int32),
        pltpu.VMEM((_B,), jnp.int32),
        pltpu.VMEM((_B,), jnp.int32),
        pltpu.VMEM((_B,), jnp.int32),
        pltpu.VMEM((_B,), jnp.int32),
        pltpu.VMEM((8, 2 * FC), jnp.float32),
        pltpu.SemaphoreType.DMA,
        pltpu.SemaphoreType.DMA,
        pltpu.SemaphoreType.DMA,
        pltpu.SemaphoreType.DMA,
        pltpu.SemaphoreType.DMA,
        pltpu.SemaphoreType.DMA,
    ],
)
def _sc_agg(pq_hbm, src4_hbm, dst_hbm, out_hbm,
            acc, rows0, rows1, rows2, srcv0, srcv1, srcv2,
            dstv0, dstv1, dstv2, zbuf, isem0, isem1, isem2,
            gsem0, gsem1, gsem2):
    cid = lax.axis_index("c")
    sid = lax.axis_index("s")
    rows = (rows0, rows1, rows2)
    srcv = (srcv0, srcv1, srcv2)
    dstv = (dstv0, dstv1, dstv2)
    isem = (isem0, isem1, isem2)
    gsem = (gsem0, gsem1, gsem2)
    zero16 = jnp.zeros((16,), jnp.float32)

    def _zero_row(i, carry):
        for j in range(2 * FC // 16):
            zbuf[i, pl.ds(j * 16, 16)] = zero16
        return carry
    lax.fori_loop(0, 8, _zero_row, 0)

    n_strips = 78 + jnp.where(sid == _SC_TILES - 1, 2, 0)

    for jchunk in range(_CPC):
        c = cid * _CPC + jchunk

        def _idx_load(j, sv, dv, sem):
            g = sid + _SC_TILES * j
            pltpu.async_copy(src4_hbm.at[pl.ds(c * E + g * _B, _B)], sv, sem)
            pltpu.async_copy(dst_hbm.at[pl.ds(g * _B, _B)], dv, sem)

        def _idx_wait(sv, dv, sem):
            pltpu.make_async_copy(src4_hbm.at[pl.ds(0, _B)], sv, sem).wait()
            pltpu.make_async_copy(dst_hbm.at[pl.ds(0, _B)], dv, sem).wait()

        def _gather(sv, rbuf, sem):
            pltpu.async_copy(pq_hbm.at[sv], rbuf, sem)

        def _gather_wait(rbuf, sem):
            pltpu.make_async_copy(pq_hbm.at[pl.ds(0, _B)], rbuf, sem).wait()

        def _scatter(rbuf, dv):
            pltpu.sync_copy(rbuf, acc.at[dv], add=True)

        def _zero_strip(i, carry):
            pltpu.sync_copy(zbuf, acc.at[pl.ds(sid * _RPT + i * 8, 8)])
            return carry
        lax.fori_loop(0, n_strips, _zero_strip, 0)

        plsc.subcore_barrier()

        _idx_load(0, srcv0, dstv0, isem0)
        _idx_load(1, srcv1, dstv1, isem1)
        _idx_load(2, srcv2, dstv2, isem2)
        _idx_wait(srcv0, dstv0, isem0)
        _gather(srcv0, rows0, gsem0)
        _idx_wait(srcv1, dstv1, isem1)
        _gather(srcv1, rows1, gsem1)

        def _step(j, b, m):
            _gather_wait(rows[b], gsem[b])
            b2 = (b + 2) % 3
            if b == 0:
                _idx_wait(srcv[b2], dstv[b2], isem[b2])
                _gather(srcv[b2], rows[b2], gsem[b2])
            else:
                @pl.when(m < _NB // 3 - 1)
                def _():
                    _idx_wait(srcv[b2], dstv[b2], isem[b2])
                    _gather(srcv[b2], rows[b2], gsem[b2])
            _scatter(rows[b], dstv[b])

            @pl.when(m < _NB // 3 - 1)
            def _():
                _idx_load(j + 3, srcv[b], dstv[b], isem[b])

        def _tri(m, carry):
            for b in range(3):
                _step(3 * m + b, b, m)
            return carry
        lax.fori_loop(0, _NB // 3, _tri, 0)

        @pl.when(sid < _NBLK - _NB * _SC_TILES)
        def _():
            g = sid + _SC_TILES * _NB
            pltpu.sync_copy(src4_hbm.at[pl.ds(c * E + g * _B, _B)], srcv0)
            pltpu.sync_copy(dst_hbm.at[pl.ds(g * _B, _B)], dstv0)
            pltpu.sync_copy(pq_hbm.at[srcv0], rows0)
            _scatter(rows0, dstv0)

        plsc.subcore_barrier()

        @pl.when(sid < _SC_TILES - 1)
        def _():
            pltpu.sync_copy(acc.at[pl.ds(sid * _RPT, _RPT)],
                            out_hbm.at[pl.ds(c * N + sid * _RPT, _RPT)])

        @pl.when(sid == _SC_TILES - 1)
        def _():
            pltpu.sync_copy(acc.at[pl.ds(sid * _RPT, _RPT_LAST)],
                            out_hbm.at[pl.ds(c * N + sid * _RPT, _RPT_LAST)])

        if jchunk + 1 < _CPC:
            plsc.subcore_barrier()


def _aggregate(pq, src4, dst):
    nd_flat = _sc_agg(pq.reshape(NCHUNK * N, 2 * FC), src4, dst)
    return nd_flat.reshape(NCHUNK, N, 2 * FC)


def kernel(x, edge_index, W_enc, b_enc, W_mlp, b_mlp, t):
    src = edge_index[0]
    dst = edge_index[1]
    src4 = (jnp.arange(NCHUNK, dtype=jnp.int32)[:, None] * N
            + src[None, :]).reshape(-1)
    h0, pq = _enc_stage(x, W_enc, b_enc, t[0])
    nd = _aggregate(pq, src4, dst)
    h, pq = _layer_stage(nd, h0, W_mlp[0], b_mlp[0], t[1],
                         first=True, last=False)
    for i in (1, 2):
        nd = _aggregate(pq, src4, dst)
        h, pq = _layer_stage(nd, h, W_mlp[i], b_mlp[i], t[i + 1],
                             first=False, last=False)
    nd = _aggregate(pq, src4, dst)
    (y,) = _layer_stage(nd, h, W_mlp[3], b_mlp[3], None,
                        first=False, last=True)
    return y
